# baseline (speedup 1.0000x reference)
import functools

import jax
import jax.numpy as jnp
import numpy as np
from jax import lax
from jax.experimental import pallas as pl
from jax.experimental.pallas import tpu as pltpu

_NCPAD = 152
_NG = _NCPAD // 8


def _hit_kernel(cc_ref, sem_ref, out_ref, code_ref):
    j = pl.program_id(1)

    @pl.when(j == 0)
    def _():
        out_ref[...] = jnp.zeros_like(out_ref)

    s = sem_ref[0]
    code_ref[...] = s[0] * 65536.0 + s[1] * 256.0 + s[2]

    def chunk(r, _):
        tile = code_ref[pl.ds(r * 8, 8), :]
        px = [jnp.broadcast_to(tile[rr:rr + 1, :], (8, 256)) for rr in range(8)]
        for g in range(_NG):
            cc_g = cc_ref[g * 8:(g + 1) * 8, :]
            acc = out_ref[0, g * 8:(g + 1) * 8, :]
            for rr in range(8):
                acc = jnp.where(px[rr] == cc_g, 1.0, acc)
            out_ref[0, g * 8:(g + 1) * 8, :] = acc
        return 0

    lax.fori_loop(0, 32, chunk, 0)


def _hit_pallas(sem, colors):
    B = sem.shape[0]
    half = B // 2
    c = colors.astype(jnp.float32)
    ccode = c[:, 0] * 65536.0 + c[:, 1] * 256.0 + c[:, 2]
    ccode = jnp.concatenate([ccode, -jnp.ones((_NCPAD - 150,), jnp.float32)])
    cc_bc = jnp.broadcast_to(ccode[:, None], (_NCPAD, 256))

    out = pl.pallas_call(
        _hit_kernel,
        grid=(2, half),
        in_specs=[
            pl.BlockSpec((_NCPAD, 256), lambda c_, j: (0, 0)),
            pl.BlockSpec((1, 3, 256, 256), lambda c_, j, h=half: (c_ * h + j, 0, 0, 0)),
        ],
        out_specs=pl.BlockSpec((1, _NCPAD, 256), lambda c_, j: (c_, 0, 0)),
        out_shape=jax.ShapeDtypeStruct((2, _NCPAD, 256), jnp.float32),
        scratch_shapes=[pltpu.VMEM((256, 256), jnp.float32)],
        compiler_params=pltpu.CompilerParams(
            dimension_semantics=("parallel", "arbitrary")),
    )(cc_bc, sem)
    return out


def _lstm_kernel(part_ref, u0f_ref, u0b_ref, w0f_ref, w0b_ref,
                 bi1_ref, bh1_ref, wi1f_ref, wi1b_ref, w1f_ref, w1b_ref,
                 lwt_ref, lb_ref, out_ref, h1f_ref, h1b_ref, h2f_ref, h2b_ref):
    T = 32

    def cell(g, c):
        i = g[:, 0:16]
        f = g[:, 16:32]
        gg = g[:, 32:48]
        o = g[:, 48:64]
        c = jax.nn.sigmoid(f) * c + jax.nn.sigmoid(i) * jnp.tanh(gg)
        h = jax.nn.sigmoid(o) * jnp.tanh(c)
        return h, c

    def l1_scan(u, w_t, h_store, fwd):
        def body(k, carry):
            h, c = carry
            g = u + jnp.dot(h, w_t, preferred_element_type=jnp.float32)
            h, c = cell(g, c)
            idx = jnp.where(fwd, k, T - 1 - k)
            h_store[pl.ds(idx, 1), :, :] = h[None]
            return (h, c)
        lax.fori_loop(0, T, body, (jnp.zeros((8, 16), jnp.float32),
                                   jnp.zeros((8, 16), jnp.float32)))

    l1_scan(u0f_ref[...], w0f_ref[...], h1f_ref, True)
    l1_scan(u0b_ref[...], w0b_ref[...], h1b_ref, False)

    def l2_scan(wi_t, w_t, bi, bh, h_store, fwd):
        def body(k, carry):
            h, c = carry
            t = jnp.where(fwd, k, T - 1 - k)
            x2 = jnp.concatenate([h1f_ref[t], h1b_ref[t]], axis=1)
            u = (jnp.dot(x2, wi_t, preferred_element_type=jnp.float32) + bi) + bh
            g = u + jnp.dot(h, w_t, preferred_element_type=jnp.float32)
            h, c = cell(g, c)
            h_store[pl.ds(t, 1), :, :] = h[None]
            return (h, c)
        lax.fori_loop(0, T, body, (jnp.zeros((8, 16), jnp.float32),
                                   jnp.zeros((8, 16), jnp.float32)))

    l2_scan(wi1f_ref[0], w1f_ref[...], bi1_ref[0], bh1_ref[0], h2f_ref, True)
    l2_scan(wi1b_ref[0], w1b_ref[...], bi1_ref[1], bh1_ref[1], h2b_ref, False)

    hit2d = jnp.maximum(part_ref[0], part_ref[1])
    hcl = jnp.minimum(jnp.sum(hit2d, axis=1, keepdims=True), 1.0)
    z = jnp.sum(hcl * lwt_ref[...], axis=0, keepdims=True)
    s_row = jnp.sum(lwt_ref[...], axis=0, keepdims=True)

    msum = jnp.sum(h2f_ref[...], axis=2) + jnp.sum(h2b_ref[...], axis=2)
    m = msum / 32.0
    v0 = m[:, 0:1]
    v1 = m[:, 1:2]
    out_ref[...] = (lb_ref[...] + v0 * s_row) + (v1 - v0) * z


def _lstm_pallas(part, lstm_params, lift_w, lift_b):
    p = lstm_params
    vcol = (jnp.arange(8) == 1).astype(jnp.float32)[:, None]

    def u0(d):
        return (vcol * p['wih0'][d][:, 0][None, :] + p['bih0'][d][None, :]) \
            + p['bhh0'][d][None, :]

    lwt = jnp.pad(lift_w.T, ((0, 2), (0, 0)))
    out = pl.pallas_call(
        _lstm_kernel,
        out_shape=jax.ShapeDtypeStruct((32, 64), jnp.float32),
        scratch_shapes=[pltpu.VMEM((32, 8, 16), jnp.float32)] * 4,
    )(part, u0(0), u0(1), p['whh0'][0].T, p['whh0'][1].T,
      p['bih1'][:, None, :], p['bhh1'][:, None, :],
      p['wih1'][0].T[None], p['wih1'][1].T[None],
      p['whh1'][0].T, p['whh1'][1].T, lwt, lift_b[None, :])
    return out


def _mm_kernel(x_ref, w_ref, b_ref, o_ref, *, nsteps, relu):
    kb = pl.program_id(1)

    @pl.when(kb == 0)
    def _():
        o_ref[...] = jnp.broadcast_to(b_ref[...], o_ref.shape)

    o_ref[...] += lax.dot_general(
        x_ref[...], w_ref[...], (((1,), (1,)), ((), ())),
        preferred_element_type=jnp.float32)

    if relu:
        @pl.when(kb == nsteps - 1)
        def _():
            o_ref[...] = jnp.maximum(o_ref[...], 0.0)


def _mm_pallas(x, w, b, relu, kb_size):
    M, K = x.shape
    N = w.shape[0]
    nk = K // kb_size
    nh = N // 2
    return pl.pallas_call(
        functools.partial(_mm_kernel, nsteps=nk, relu=relu),
        grid=(2, nk),
        in_specs=[
            pl.BlockSpec((M, kb_size), lambda p, k: (0, k)),
            pl.BlockSpec((nh, kb_size), lambda p, k: (p, k)),
            pl.BlockSpec((1, nh), lambda p, k: (0, p)),
        ],
        out_specs=pl.BlockSpec((M, nh), lambda p, k: (0, p)),
        out_shape=jax.ShapeDtypeStruct((M, N), jnp.float32),
        compiler_params=pltpu.CompilerParams(
            dimension_semantics=("parallel", "arbitrary")),
    )(x, w, b)


def _fuse_kernel(x1_ref, x2_ref, w1_ref, w2_ref, b_ref, o_ref):
    y = lax.dot_general(x1_ref[...], w1_ref[...], (((1,), (1,)), ((), ())),
                        preferred_element_type=jnp.float32)
    y += lax.dot_general(x2_ref[...], w2_ref[...], (((1,), (1,)), ((), ())),
                         preferred_element_type=jnp.float32)
    o_ref[...] = y + b_ref[...]


def _fc_head_pallas(featc, featb, cp, bp, fc1_w, fc1_b):
    def fc_stack(p, x, n3_pad):
        x = _mm_pallas(x, p['f1w'], p['f1b'][None, :], True, 1024)
        x = _mm_pallas(x, p['f2w'], p['f2b'][None, :], True, 1024)
        w3 = jnp.pad(p['f3w'], ((0, n3_pad - p['f3w'].shape[0]), (0, 0)))
        b3 = jnp.pad(p['f3b'], (0, n3_pad - p['f3b'].shape[0]))
        return _mm_pallas(x, w3, b3[None, :], False, 1024)

    cf = fc_stack(cp, featc, 512)
    bf = fc_stack(bp, featb, 1024)
    wc = jnp.pad(fc1_w[:, :365], ((0, 0), (0, 512 - 365)))
    wb = jnp.pad(fc1_w[:, 365:], ((0, 0), (0, 1024 - 1000)))
    fuse = pl.pallas_call(
        _fuse_kernel,
        out_shape=jax.ShapeDtypeStruct((32, 512), jnp.float32),
    )(cf, bf, wc, wb, fc1_b[None, :])
    return fuse


def _convT_kernel(x_ref, w_ref, b_ref, o_ref, p_ref, *, cin, offs, nl):
    for i, off in enumerate(offs):
        p_ref[i * cin:(i + 1) * cin, :] = x_ref[0, 0, :, pl.ds(off, nl)]
    acc = jnp.dot(w_ref[0], p_ref[...], preferred_element_type=jnp.float32)
    o_ref[0, 0] = jnp.maximum(acc + b_ref[0], 0.0)


def _convT_pallas(x4, w, b, offs, nl):
    _, _, cin, spad = x4.shape
    cout, kg = w.shape[1], w.shape[2]
    return pl.pallas_call(
        functools.partial(_convT_kernel, cin=cin, offs=offs, nl=nl),
        grid=(2, 32),
        in_specs=[
            pl.BlockSpec((1, 1, cin, spad), lambda c, j: (c, j, 0, 0)),
            pl.BlockSpec((1, cout, kg), lambda c, j: (c, 0, 0)),
            pl.BlockSpec((1, cout, 1), lambda c, j: (c, 0, 0)),
        ],
        out_specs=pl.BlockSpec((1, 1, cout, nl), lambda c, j: (c, j, 0, 0)),
        out_shape=jax.ShapeDtypeStruct((2, 32, cout, nl), jnp.float32),
        scratch_shapes=[pltpu.VMEM((kg, nl), jnp.float32)],
        compiler_params=pltpu.CompilerParams(
            dimension_semantics=("parallel", "arbitrary")),
    )(x4, w, b)


def _pack_w(wc, wb, taps):
    def one(w):
        return jnp.concatenate([w[:, :, kh, kw] for kh, kw in taps], axis=1)
    return jnp.stack([one(wc), one(wb)])


def _lanes_to_grid(y, g, hv, pool, pad):
    c = y.shape[2]
    y = jnp.pad(y, ((0, 0), (0, 0), (0, 0), (0, g * g - y.shape[3])))
    y = y.reshape(2, 32, c, g, g)[:, :, :, :hv, :hv]
    if pool:
        y = lax.reduce_window(y, -jnp.inf, lax.max,
                              (1, 1, 1, 3, 3), (1, 1, 1, 2, 2), 'VALID')
    if pad:
        y = jnp.pad(y, ((0, 0), (0, 0), (0, 0), (pad, pad), (pad, pad)))
    s = y.shape[3]
    y = y.reshape(2, 32, c, s * s)
    lpad = (-(s * s) % 8) or 0
    return jnp.pad(y, ((0, 0), (0, 0), (0, 0), (0, lpad))), s


def _alexnet_convs_pallas(cp, bp, context, body):
    def s2d(x):
        xp = jnp.pad(x, ((0, 0), (0, 0), (2, 2), (2, 2)))
        xp = xp.reshape(32, 3, 65, 4, 65, 4)
        xp = xp.transpose(0, 3, 5, 1, 2, 4)
        return xp.reshape(32, 48, 4225)

    x1 = jnp.pad(jnp.stack([s2d(context), s2d(body)]),
                 ((0, 0), (0, 0), (0, 0), (0, 7)))

    def w1pack(w):
        wp = jnp.pad(w, ((0, 0), (0, 0), (0, 1), (0, 1)))
        wr = wp.reshape(64, 3, 3, 4, 3, 4)
        wt = wr.transpose(2, 4, 0, 3, 5, 1)
        return wt.reshape(9, 64, 48).transpose(1, 0, 2).reshape(64, 432)

    w1 = jnp.stack([w1pack(cp['c1w']), w1pack(bp['c1w'])])
    b1 = jnp.stack([cp['c1b'][:, None], bp['c1b'][:, None]])
    offs1 = [a * 65 + b for a in range(3) for b in range(3)]
    y = _convT_pallas(x1, w1, b1, offs1, 4096)

    x2, _ = _lanes_to_grid(y, 65, 63, True, 2)
    taps2 = [(kh, kw) for kh in range(5) for kw in range(5)]
    w2 = _pack_w(cp['c2w'], bp['c2w'], taps2)
    b2 = jnp.stack([cp['c2b'][:, None], bp['c2b'][:, None]])
    offs2 = [kh * 35 + kw for kh, kw in taps2]
    y = _convT_pallas(x2, w2, b2, offs2, 1088)

    x3, _ = _lanes_to_grid(y, 35, 31, True, 1)
    taps3 = [(kh, kw) for kh in range(3) for kw in range(3)]
    offs3 = [kh * 17 + kw for kh, kw in taps3]
    w3 = _pack_w(cp['c3w'], bp['c3w'], taps3)
    b3 = jnp.stack([cp['c3b'][:, None], bp['c3b'][:, None]])
    y = _convT_pallas(x3, w3, b3, offs3, 256)

    x4, _ = _lanes_to_grid(y, 17, 15, False, 1)
    w4 = _pack_w(cp['c4w'], bp['c4w'], taps3)
    b4 = jnp.stack([cp['c4b'][:, None], bp['c4b'][:, None]])
    y = _convT_pallas(x4, w4, b4, offs3, 256)

    x5, _ = _lanes_to_grid(y, 17, 15, False, 1)
    w5 = _pack_w(cp['c5w'], bp['c5w'], taps3)
    b5 = jnp.stack([cp['c5b'][:, None], bp['c5b'][:, None]])
    y = _convT_pallas(x5, w5, b5, offs3, 256)

    y = jnp.pad(y, ((0, 0), (0, 0), (0, 0), (0, 33)))
    y = y.reshape(2, 32, 256, 17, 17)[:, :, :, :15, :15]
    y = lax.reduce_window(y, -jnp.inf, lax.max,
                          (1, 1, 1, 3, 3), (1, 1, 1, 2, 2), 'VALID')
    y = lax.reduce_window(y, 0.0, lax.add,
                          (1, 1, 1, 2, 2), (1, 1, 1, 1, 1), 'VALID') * 0.25
    feat = y.reshape(2, 32, 9216)
    return feat[0], feat[1]


def kernel(x, colors, ctx_params, body_params, lstm_params, fc1_w, fc1_b, lift_w, lift_b):
    context = x[:, :, :256, :]
    body = x[:, :, 256:512, :]
    sem = x[:, :, 512:768, :]

    part = _hit_pallas(sem, colors)
    feat_sem = _lstm_pallas(part, lstm_params, lift_w, lift_b)

    featc, featb = _alexnet_convs_pallas(ctx_params, body_params, context, body)
    fuse = _fc_head_pallas(featc, featb, ctx_params, body_params, fc1_w, fc1_b)
    return fuse, feat_sem


# speedup vs baseline: 4.3376x; 4.3376x over previous
import functools

import jax
import jax.numpy as jnp
import numpy as np
from jax import lax
from jax.experimental import pallas as pl
from jax.experimental.pallas import tpu as pltpu

_NCPAD = 152
_NG = _NCPAD // 8


def _hit_kernel(cc_ref, sem_ref, out_ref, code_ref):
    j = pl.program_id(1)

    @pl.when(j == 0)
    def _():
        out_ref[...] = jnp.zeros_like(out_ref)

    s = sem_ref[0]
    code_ref[...] = s[0] * 65536.0 + s[1] * 256.0 + s[2]

    def chunk(r, _):
        tile = code_ref[pl.ds(r * 8, 8), :]
        px = [jnp.broadcast_to(tile[rr:rr + 1, :], (8, 256)) for rr in range(8)]
        for g in range(_NG):
            cc_g = cc_ref[g * 8:(g + 1) * 8, :]
            acc = out_ref[0, g * 8:(g + 1) * 8, :]
            for rr in range(8):
                acc = jnp.where(px[rr] == cc_g, 1.0, acc)
            out_ref[0, g * 8:(g + 1) * 8, :] = acc
        return 0

    lax.fori_loop(0, 32, chunk, 0)


def _hit_pallas(sem, colors):
    B = sem.shape[0]
    half = B // 2
    c = colors.astype(jnp.float32)
    ccode = c[:, 0] * 65536.0 + c[:, 1] * 256.0 + c[:, 2]
    ccode = jnp.concatenate([ccode, -jnp.ones((_NCPAD - 150,), jnp.float32)])
    cc_bc = jnp.broadcast_to(ccode[:, None], (_NCPAD, 256))

    out = pl.pallas_call(
        _hit_kernel,
        grid=(2, half),
        in_specs=[
            pl.BlockSpec((_NCPAD, 256), lambda c_, j: (0, 0)),
            pl.BlockSpec((1, 3, 256, 256), lambda c_, j, h=half: (c_ * h + j, 0, 0, 0)),
        ],
        out_specs=pl.BlockSpec((1, _NCPAD, 256), lambda c_, j: (c_, 0, 0)),
        out_shape=jax.ShapeDtypeStruct((2, _NCPAD, 256), jnp.float32),
        scratch_shapes=[pltpu.VMEM((256, 256), jnp.float32)],
        compiler_params=pltpu.CompilerParams(
            dimension_semantics=("parallel", "arbitrary")),
    )(cc_bc, sem)
    return out


def _lstm_kernel(part_ref, u0f_ref, u0b_ref, w0f_ref, w0b_ref,
                 bi1_ref, bh1_ref, wi1f_ref, wi1b_ref, w1f_ref, w1b_ref,
                 lwt_ref, lb_ref, out_ref, h1f_ref, h1b_ref, h2f_ref, h2b_ref):
    T = 32

    def cell(g, c):
        i = g[:, 0:16]
        f = g[:, 16:32]
        gg = g[:, 32:48]
        o = g[:, 48:64]
        c = jax.nn.sigmoid(f) * c + jax.nn.sigmoid(i) * jnp.tanh(gg)
        h = jax.nn.sigmoid(o) * jnp.tanh(c)
        return h, c

    def l1_scan(u, w_t, h_store, fwd):
        def body(k, carry):
            h, c = carry
            g = u + jnp.dot(h, w_t, preferred_element_type=jnp.float32)
            h, c = cell(g, c)
            idx = jnp.where(fwd, k, T - 1 - k)
            h_store[pl.ds(idx, 1), :, :] = h[None]
            return (h, c)
        lax.fori_loop(0, T, body, (jnp.zeros((8, 16), jnp.float32),
                                   jnp.zeros((8, 16), jnp.float32)))

    l1_scan(u0f_ref[...], w0f_ref[...], h1f_ref, True)
    l1_scan(u0b_ref[...], w0b_ref[...], h1b_ref, False)

    def l2_scan(wi_t, w_t, bi, bh, h_store, fwd):
        def body(k, carry):
            h, c = carry
            t = jnp.where(fwd, k, T - 1 - k)
            x2 = jnp.concatenate([h1f_ref[t], h1b_ref[t]], axis=1)
            u = (jnp.dot(x2, wi_t, preferred_element_type=jnp.float32) + bi) + bh
            g = u + jnp.dot(h, w_t, preferred_element_type=jnp.float32)
            h, c = cell(g, c)
            h_store[pl.ds(t, 1), :, :] = h[None]
            return (h, c)
        lax.fori_loop(0, T, body, (jnp.zeros((8, 16), jnp.float32),
                                   jnp.zeros((8, 16), jnp.float32)))

    l2_scan(wi1f_ref[0], w1f_ref[...], bi1_ref[0], bh1_ref[0], h2f_ref, True)
    l2_scan(wi1b_ref[0], w1b_ref[...], bi1_ref[1], bh1_ref[1], h2b_ref, False)

    hit2d = jnp.maximum(part_ref[0], part_ref[1])
    hcl = jnp.minimum(jnp.sum(hit2d, axis=1, keepdims=True), 1.0)
    z = jnp.sum(hcl * lwt_ref[...], axis=0, keepdims=True)
    s_row = jnp.sum(lwt_ref[...], axis=0, keepdims=True)

    msum = jnp.sum(h2f_ref[...], axis=2) + jnp.sum(h2b_ref[...], axis=2)
    m = msum / 32.0
    v0 = m[:, 0:1]
    v1 = m[:, 1:2]
    out_ref[...] = (lb_ref[...] + v0 * s_row) + (v1 - v0) * z


def _lstm_pallas(part, lstm_params, lift_w, lift_b):
    p = lstm_params
    vcol = (jnp.arange(8) == 1).astype(jnp.float32)[:, None]

    def u0(d):
        return (vcol * p['wih0'][d][:, 0][None, :] + p['bih0'][d][None, :]) \
            + p['bhh0'][d][None, :]

    lwt = jnp.pad(lift_w.T, ((0, 2), (0, 0)))
    out = pl.pallas_call(
        _lstm_kernel,
        out_shape=jax.ShapeDtypeStruct((32, 64), jnp.float32),
        scratch_shapes=[pltpu.VMEM((32, 8, 16), jnp.float32)] * 4,
    )(part, u0(0), u0(1), p['whh0'][0].T, p['whh0'][1].T,
      p['bih1'][:, None, :], p['bhh1'][:, None, :],
      p['wih1'][0].T[None], p['wih1'][1].T[None],
      p['whh1'][0].T, p['whh1'][1].T, lwt, lift_b[None, :])
    return out


def _mm_kernel(x_ref, w_ref, b_ref, o_ref, *, nsteps, relu):
    kb = pl.program_id(1)

    @pl.when(kb == 0)
    def _():
        o_ref[...] = jnp.broadcast_to(b_ref[...], o_ref.shape)

    o_ref[...] += lax.dot_general(
        x_ref[...], w_ref[...], (((1,), (1,)), ((), ())),
        preferred_element_type=jnp.float32)

    if relu:
        @pl.when(kb == nsteps - 1)
        def _():
            o_ref[...] = jnp.maximum(o_ref[...], 0.0)


def _mm_pallas(x, w, b, relu, kb_size):
    M, K = x.shape
    N = w.shape[0]
    nk = K // kb_size
    nh = N // 2
    return pl.pallas_call(
        functools.partial(_mm_kernel, nsteps=nk, relu=relu),
        grid=(2, nk),
        in_specs=[
            pl.BlockSpec((M, kb_size), lambda p, k: (0, k)),
            pl.BlockSpec((nh, kb_size), lambda p, k: (p, k)),
            pl.BlockSpec((1, nh), lambda p, k: (0, p)),
        ],
        out_specs=pl.BlockSpec((M, nh), lambda p, k: (0, p)),
        out_shape=jax.ShapeDtypeStruct((M, N), jnp.float32),
        compiler_params=pltpu.CompilerParams(
            dimension_semantics=("parallel", "arbitrary")),
    )(x, w, b)


def _fuse_kernel(x1_ref, x2_ref, w1_ref, w2_ref, b_ref, o_ref):
    y = lax.dot_general(x1_ref[...], w1_ref[...], (((1,), (1,)), ((), ())),
                        preferred_element_type=jnp.float32)
    y += lax.dot_general(x2_ref[...], w2_ref[...], (((1,), (1,)), ((), ())),
                         preferred_element_type=jnp.float32)
    o_ref[...] = y + b_ref[...]


def _fc_head_pallas(featc, featb, cp, bp, fc1_w, fc1_b):
    def fc_stack(p, x, n3_pad):
        x = _mm_pallas(x, p['f1w'], p['f1b'][None, :], True, 1024)
        x = _mm_pallas(x, p['f2w'], p['f2b'][None, :], True, 1024)
        w3 = jnp.pad(p['f3w'], ((0, n3_pad - p['f3w'].shape[0]), (0, 0)))
        b3 = jnp.pad(p['f3b'], (0, n3_pad - p['f3b'].shape[0]))
        return _mm_pallas(x, w3, b3[None, :], False, 1024)

    cf = fc_stack(cp, featc, 512)
    bf = fc_stack(bp, featb, 1024)
    wc = jnp.pad(fc1_w[:, :365], ((0, 0), (0, 512 - 365)))
    wb = jnp.pad(fc1_w[:, 365:], ((0, 0), (0, 1024 - 1000)))
    fuse = pl.pallas_call(
        _fuse_kernel,
        out_shape=jax.ShapeDtypeStruct((32, 512), jnp.float32),
    )(cf, bf, wc, wb, fc1_b[None, :])
    return fuse


def _convT_kernel(x_ref, w_ref, b_ref, o_ref, p_ref, *, cin, offs, nl):
    for i, off in enumerate(offs):
        p_ref[i * cin:(i + 1) * cin, :] = x_ref[0, 0, :, pl.ds(off, nl)]
    acc = jnp.dot(w_ref[0], p_ref[...], preferred_element_type=jnp.float32)
    o_ref[0, 0] = jnp.maximum(acc + b_ref[0], 0.0)


def _convT_pallas(x4, w, b, offs, nl):
    _, _, cin, spad = x4.shape
    cout, kg = w.shape[1], w.shape[2]
    return pl.pallas_call(
        functools.partial(_convT_kernel, cin=cin, offs=offs, nl=nl),
        grid=(2, 32),
        in_specs=[
            pl.BlockSpec((1, 1, cin, spad), lambda c, j: (c, j, 0, 0)),
            pl.BlockSpec((1, cout, kg), lambda c, j: (c, 0, 0)),
            pl.BlockSpec((1, cout, 1), lambda c, j: (c, 0, 0)),
        ],
        out_specs=pl.BlockSpec((1, 1, cout, nl), lambda c, j: (c, j, 0, 0)),
        out_shape=jax.ShapeDtypeStruct((2, 32, cout, nl), jnp.float32),
        scratch_shapes=[pltpu.VMEM((kg, nl), jnp.float32)],
        compiler_params=pltpu.CompilerParams(
            dimension_semantics=("parallel", "arbitrary")),
    )(x4, w, b)


def _pack_w(wc, wb, taps):
    def one(w):
        return jnp.concatenate([w[:, :, kh, kw] for kh, kw in taps], axis=1)
    return jnp.stack([one(wc), one(wb)])


def _lanes_to_grid(y, g, hv, pool, pad):
    c = y.shape[2]
    y = jnp.pad(y, ((0, 0), (0, 0), (0, 0), (0, g * g - y.shape[3])))
    y = y.reshape(2, 32, c, g, g)[:, :, :, :hv, :hv]
    if pool:
        y = lax.reduce_window(y, -jnp.inf, lax.max,
                              (1, 1, 1, 3, 3), (1, 1, 1, 2, 2), 'VALID')
    if pad:
        y = jnp.pad(y, ((0, 0), (0, 0), (0, 0), (pad, pad), (pad, pad)))
    s = y.shape[3]
    y = y.reshape(2, 32, c, s * s)
    lpad = (-(s * s) % 8) or 0
    return jnp.pad(y, ((0, 0), (0, 0), (0, 0), (0, lpad))), s


def _alexnet_convs_pallas(cp, bp, context, body):
    def s2d(x):
        xp = jnp.pad(x, ((0, 0), (0, 0), (2, 2), (2, 2)))
        xp = xp.reshape(32, 3, 65, 4, 65, 4)
        xp = xp.transpose(0, 3, 5, 1, 2, 4)
        return xp.reshape(32, 48, 4225)

    x1 = jnp.pad(jnp.stack([s2d(context), s2d(body)]),
                 ((0, 0), (0, 0), (0, 0), (0, 7)))

    def w1pack(w):
        wp = jnp.pad(w, ((0, 0), (0, 0), (0, 1), (0, 1)))
        wr = wp.reshape(64, 3, 3, 4, 3, 4)
        wt = wr.transpose(2, 4, 0, 3, 5, 1)
        return wt.reshape(9, 64, 48).transpose(1, 0, 2).reshape(64, 432)

    w1 = jnp.stack([w1pack(cp['c1w']), w1pack(bp['c1w'])])
    b1 = jnp.stack([cp['c1b'][:, None], bp['c1b'][:, None]])
    offs1 = [a * 65 + b for a in range(3) for b in range(3)]
    y = _convT_pallas(x1, w1, b1, offs1, 4096)

    x2, _ = _lanes_to_grid(y, 65, 63, True, 2)
    taps2 = [(kh, kw) for kh in range(5) for kw in range(5)]
    w2 = _pack_w(cp['c2w'], bp['c2w'], taps2)
    b2 = jnp.stack([cp['c2b'][:, None], bp['c2b'][:, None]])
    offs2 = [kh * 35 + kw for kh, kw in taps2]
    y = _convT_pallas(x2, w2, b2, offs2, 1088)

    x3, _ = _lanes_to_grid(y, 35, 31, True, 1)
    taps3 = [(kh, kw) for kh in range(3) for kw in range(3)]
    offs3 = [kh * 17 + kw for kh, kw in taps3]
    w3 = _pack_w(cp['c3w'], bp['c3w'], taps3)
    b3 = jnp.stack([cp['c3b'][:, None], bp['c3b'][:, None]])
    y = _convT_pallas(x3, w3, b3, offs3, 256)

    x4, _ = _lanes_to_grid(y, 17, 15, False, 1)
    w4 = _pack_w(cp['c4w'], bp['c4w'], taps3)
    b4 = jnp.stack([cp['c4b'][:, None], bp['c4b'][:, None]])
    y = _convT_pallas(x4, w4, b4, offs3, 256)

    x5, _ = _lanes_to_grid(y, 17, 15, False, 1)
    w5 = _pack_w(cp['c5w'], bp['c5w'], taps3)
    b5 = jnp.stack([cp['c5b'][:, None], bp['c5b'][:, None]])
    y = _convT_pallas(x5, w5, b5, offs3, 256)

    y = jnp.pad(y, ((0, 0), (0, 0), (0, 0), (0, 33)))
    y = y.reshape(2, 32, 256, 17, 17)[:, :, :, :15, :15]
    y = lax.reduce_window(y, -jnp.inf, lax.max,
                          (1, 1, 1, 3, 3), (1, 1, 1, 2, 2), 'VALID')
    y = lax.reduce_window(y, 0.0, lax.add,
                          (1, 1, 1, 2, 2), (1, 1, 1, 1, 1), 'VALID') * 0.25
    feat = y.reshape(2, 32, 9216)
    return feat[0], feat[1]


def _conv(x, w, b, stride, pad):
    y = lax.conv_general_dilated(x, w, (stride, stride), [(pad, pad), (pad, pad)],
                                 dimension_numbers=('NCHW', 'OIHW', 'NCHW'))
    return y + b[None, :, None, None]


def _maxpool(x):
    return lax.reduce_window(x, -jnp.inf, lax.max, (1, 1, 3, 3), (1, 1, 2, 2), 'VALID')


def _alexnet_features(p, x):
    x = jax.nn.relu(_conv(x, p['c1w'], p['c1b'], 4, 2)); x = _maxpool(x)
    x = jax.nn.relu(_conv(x, p['c2w'], p['c2b'], 1, 2)); x = _maxpool(x)
    x = jax.nn.relu(_conv(x, p['c3w'], p['c3b'], 1, 1))
    x = jax.nn.relu(_conv(x, p['c4w'], p['c4b'], 1, 1))
    x = jax.nn.relu(_conv(x, p['c5w'], p['c5b'], 1, 1)); x = _maxpool(x)
    x = lax.reduce_window(x, 0.0, lax.add, (1, 1, 2, 2), (1, 1, 1, 1), 'VALID') * 0.25
    return x.reshape(x.shape[0], -1)


def kernel(x, colors, ctx_params, body_params, lstm_params, fc1_w, fc1_b, lift_w, lift_b):
    context = x[:, :, :256, :]
    body = x[:, :, 256:512, :]
    sem = x[:, :, 512:768, :]

    part = _hit_pallas(sem, colors)
    feat_sem = _lstm_pallas(part, lstm_params, lift_w, lift_b)

    featc = _alexnet_features(ctx_params, context)
    featb = _alexnet_features(body_params, body)
    fuse = _fc_head_pallas(featc, featb, ctx_params, body_params, fc1_w, fc1_b)
    return fuse, feat_sem


# speedup vs baseline: 4.3438x; 1.0014x over previous
import functools

import jax
import jax.numpy as jnp
import numpy as np
from jax import lax
from jax.experimental import pallas as pl
from jax.experimental.pallas import tpu as pltpu

_NCPAD = 152
_NG = _NCPAD // 8


def _hit_kernel(cc_ref, sem_ref, out_ref, code_ref):
    j = pl.program_id(1)

    @pl.when(j == 0)
    def _():
        out_ref[...] = jnp.zeros_like(out_ref)

    s = sem_ref[0]
    code_ref[...] = s[0] * 65536.0 + s[1] * 256.0 + s[2]

    def chunk(r, _):
        tile = code_ref[pl.ds(r * 8, 8), :]
        px = [jnp.broadcast_to(tile[rr:rr + 1, :], (8, 256)) for rr in range(8)]
        for g in range(_NG):
            cc_g = cc_ref[g * 8:(g + 1) * 8, :]
            acc = out_ref[0, g * 8:(g + 1) * 8, :]
            for rr in range(8):
                acc = jnp.where(px[rr] == cc_g, 1.0, acc)
            out_ref[0, g * 8:(g + 1) * 8, :] = acc
        return 0

    lax.fori_loop(0, 32, chunk, 0)


def _hit_pallas(sem, colors):
    B = sem.shape[0]
    half = B // 2
    c = colors.astype(jnp.float32)
    ccode = c[:, 0] * 65536.0 + c[:, 1] * 256.0 + c[:, 2]
    ccode = jnp.concatenate([ccode, -jnp.ones((_NCPAD - 150,), jnp.float32)])
    cc_bc = jnp.broadcast_to(ccode[:, None], (_NCPAD, 256))

    out = pl.pallas_call(
        _hit_kernel,
        grid=(2, half),
        in_specs=[
            pl.BlockSpec((_NCPAD, 256), lambda c_, j: (0, 0)),
            pl.BlockSpec((1, 3, 256, 256), lambda c_, j, h=half: (c_ * h + j, 0, 0, 0)),
        ],
        out_specs=pl.BlockSpec((1, _NCPAD, 256), lambda c_, j: (c_, 0, 0)),
        out_shape=jax.ShapeDtypeStruct((2, _NCPAD, 256), jnp.float32),
        scratch_shapes=[pltpu.VMEM((256, 256), jnp.float32)],
        compiler_params=pltpu.CompilerParams(
            dimension_semantics=("parallel", "arbitrary")),
    )(cc_bc, sem)
    return out


def _lstm_kernel(part_ref, u0f_ref, u0b_ref, w0f_ref, w0b_ref,
                 bi1_ref, bh1_ref, wi1f_ref, wi1b_ref, w1f_ref, w1b_ref,
                 lwt_ref, lb_ref, out_ref, h1f_ref, h1b_ref, h2f_ref, h2b_ref):
    T = 32

    def cell(g, c):
        i = g[:, 0:16]
        f = g[:, 16:32]
        gg = g[:, 32:48]
        o = g[:, 48:64]
        c = jax.nn.sigmoid(f) * c + jax.nn.sigmoid(i) * jnp.tanh(gg)
        h = jax.nn.sigmoid(o) * jnp.tanh(c)
        return h, c

    def l1_scan(u, w_t, h_store, fwd):
        def body(k, carry):
            h, c = carry
            g = u + jnp.dot(h, w_t, preferred_element_type=jnp.float32)
            h, c = cell(g, c)
            idx = jnp.where(fwd, k, T - 1 - k)
            h_store[pl.ds(idx, 1), :, :] = h[None]
            return (h, c)
        lax.fori_loop(0, T, body, (jnp.zeros((8, 16), jnp.float32),
                                   jnp.zeros((8, 16), jnp.float32)))

    l1_scan(u0f_ref[...], w0f_ref[...], h1f_ref, True)
    l1_scan(u0b_ref[...], w0b_ref[...], h1b_ref, False)

    def l2_scan(wi_t, w_t, bi, bh, h_store, fwd):
        def body(k, carry):
            h, c = carry
            t = jnp.where(fwd, k, T - 1 - k)
            x2 = jnp.concatenate([h1f_ref[t], h1b_ref[t]], axis=1)
            u = (jnp.dot(x2, wi_t, preferred_element_type=jnp.float32) + bi) + bh
            g = u + jnp.dot(h, w_t, preferred_element_type=jnp.float32)
            h, c = cell(g, c)
            h_store[pl.ds(t, 1), :, :] = h[None]
            return (h, c)
        lax.fori_loop(0, T, body, (jnp.zeros((8, 16), jnp.float32),
                                   jnp.zeros((8, 16), jnp.float32)))

    l2_scan(wi1f_ref[0], w1f_ref[...], bi1_ref[0], bh1_ref[0], h2f_ref, True)
    l2_scan(wi1b_ref[0], w1b_ref[...], bi1_ref[1], bh1_ref[1], h2b_ref, False)

    hit2d = jnp.maximum(part_ref[0], part_ref[1])
    hcl = jnp.minimum(jnp.sum(hit2d, axis=1, keepdims=True), 1.0)
    z = jnp.sum(hcl * lwt_ref[...], axis=0, keepdims=True)
    s_row = jnp.sum(lwt_ref[...], axis=0, keepdims=True)

    msum = jnp.sum(h2f_ref[...], axis=2) + jnp.sum(h2b_ref[...], axis=2)
    m = msum / 32.0
    v0 = m[:, 0:1]
    v1 = m[:, 1:2]
    out_ref[...] = (lb_ref[...] + v0 * s_row) + (v1 - v0) * z


def _lstm_pallas(part, lstm_params, lift_w, lift_b):
    p = lstm_params
    vcol = (jnp.arange(8) == 1).astype(jnp.float32)[:, None]

    def u0(d):
        return (vcol * p['wih0'][d][:, 0][None, :] + p['bih0'][d][None, :]) \
            + p['bhh0'][d][None, :]

    lwt = jnp.pad(lift_w.T, ((0, 2), (0, 0)))
    out = pl.pallas_call(
        _lstm_kernel,
        out_shape=jax.ShapeDtypeStruct((32, 64), jnp.float32),
        scratch_shapes=[pltpu.VMEM((32, 8, 16), jnp.float32)] * 4,
    )(part, u0(0), u0(1), p['whh0'][0].T, p['whh0'][1].T,
      p['bih1'][:, None, :], p['bhh1'][:, None, :],
      p['wih1'][0].T[None], p['wih1'][1].T[None],
      p['whh1'][0].T, p['whh1'][1].T, lwt, lift_b[None, :])
    return out


def _mm_kernel(x_ref, w_ref, b_ref, o_ref, *, nsteps, relu):
    kb = pl.program_id(1)

    @pl.when(kb == 0)
    def _():
        o_ref[...] = jnp.broadcast_to(b_ref[...], o_ref.shape)

    o_ref[...] += lax.dot_general(
        x_ref[...], w_ref[...], (((1,), (1,)), ((), ())),
        preferred_element_type=jnp.float32)

    if relu:
        @pl.when(kb == nsteps - 1)
        def _():
            o_ref[...] = jnp.maximum(o_ref[...], 0.0)


def _mm_pallas(x, w, b, relu, kb_size):
    M, K = x.shape
    N = w.shape[0]
    nk = K // kb_size
    nh = N // 2
    return pl.pallas_call(
        functools.partial(_mm_kernel, nsteps=nk, relu=relu),
        grid=(2, nk),
        in_specs=[
            pl.BlockSpec((M, kb_size), lambda p, k: (0, k)),
            pl.BlockSpec((nh, kb_size), lambda p, k: (p, k)),
            pl.BlockSpec((1, nh), lambda p, k: (0, p)),
        ],
        out_specs=pl.BlockSpec((M, nh), lambda p, k: (0, p)),
        out_shape=jax.ShapeDtypeStruct((M, N), jnp.float32),
        compiler_params=pltpu.CompilerParams(
            dimension_semantics=("parallel", "arbitrary")),
    )(x, w, b)


def _fuse_kernel(x1_ref, x2_ref, w1_ref, w2_ref, b_ref, o_ref):
    y = lax.dot_general(x1_ref[...], w1_ref[...], (((1,), (1,)), ((), ())),
                        preferred_element_type=jnp.float32)
    y += lax.dot_general(x2_ref[...], w2_ref[...], (((1,), (1,)), ((), ())),
                         preferred_element_type=jnp.float32)
    o_ref[...] = y + b_ref[...]


def _fc_head_pallas(featc, featb, cp, bp, fc1_w, fc1_b):
    def fc_stack(p, x, n3_pad):
        x = _mm_pallas(x, p['f1w'], p['f1b'][None, :], True, 2304)
        x = _mm_pallas(x, p['f2w'], p['f2b'][None, :], True, 2048)
        w3 = jnp.pad(p['f3w'], ((0, n3_pad - p['f3w'].shape[0]), (0, 0)))
        b3 = jnp.pad(p['f3b'], (0, n3_pad - p['f3b'].shape[0]))
        return _mm_pallas(x, w3, b3[None, :], False, 2048)

    cf = fc_stack(cp, featc, 512)
    bf = fc_stack(bp, featb, 1024)
    wc = jnp.pad(fc1_w[:, :365], ((0, 0), (0, 512 - 365)))
    wb = jnp.pad(fc1_w[:, 365:], ((0, 0), (0, 1024 - 1000)))
    fuse = pl.pallas_call(
        _fuse_kernel,
        out_shape=jax.ShapeDtypeStruct((32, 512), jnp.float32),
    )(cf, bf, wc, wb, fc1_b[None, :])
    return fuse


def _convT_kernel(x_ref, w_ref, b_ref, o_ref, p_ref, *, cin, offs, nl):
    for i, off in enumerate(offs):
        p_ref[i * cin:(i + 1) * cin, :] = x_ref[0, 0, :, pl.ds(off, nl)]
    acc = jnp.dot(w_ref[0], p_ref[...], preferred_element_type=jnp.float32)
    o_ref[0, 0] = jnp.maximum(acc + b_ref[0], 0.0)


def _convT_pallas(x4, w, b, offs, nl):
    _, _, cin, spad = x4.shape
    cout, kg = w.shape[1], w.shape[2]
    return pl.pallas_call(
        functools.partial(_convT_kernel, cin=cin, offs=offs, nl=nl),
        grid=(2, 32),
        in_specs=[
            pl.BlockSpec((1, 1, cin, spad), lambda c, j: (c, j, 0, 0)),
            pl.BlockSpec((1, cout, kg), lambda c, j: (c, 0, 0)),
            pl.BlockSpec((1, cout, 1), lambda c, j: (c, 0, 0)),
        ],
        out_specs=pl.BlockSpec((1, 1, cout, nl), lambda c, j: (c, j, 0, 0)),
        out_shape=jax.ShapeDtypeStruct((2, 32, cout, nl), jnp.float32),
        scratch_shapes=[pltpu.VMEM((kg, nl), jnp.float32)],
        compiler_params=pltpu.CompilerParams(
            dimension_semantics=("parallel", "arbitrary")),
    )(x4, w, b)


def _pack_w(wc, wb, taps):
    def one(w):
        return jnp.concatenate([w[:, :, kh, kw] for kh, kw in taps], axis=1)
    return jnp.stack([one(wc), one(wb)])


def _lanes_to_grid(y, g, hv, pool, pad):
    c = y.shape[2]
    y = jnp.pad(y, ((0, 0), (0, 0), (0, 0), (0, g * g - y.shape[3])))
    y = y.reshape(2, 32, c, g, g)[:, :, :, :hv, :hv]
    if pool:
        y = lax.reduce_window(y, -jnp.inf, lax.max,
                              (1, 1, 1, 3, 3), (1, 1, 1, 2, 2), 'VALID')
    if pad:
        y = jnp.pad(y, ((0, 0), (0, 0), (0, 0), (pad, pad), (pad, pad)))
    s = y.shape[3]
    y = y.reshape(2, 32, c, s * s)
    lpad = (-(s * s) % 8) or 0
    return jnp.pad(y, ((0, 0), (0, 0), (0, 0), (0, lpad))), s


def _alexnet_convs_pallas(cp, bp, context, body):
    def s2d(x):
        xp = jnp.pad(x, ((0, 0), (0, 0), (2, 2), (2, 2)))
        xp = xp.reshape(32, 3, 65, 4, 65, 4)
        xp = xp.transpose(0, 3, 5, 1, 2, 4)
        return xp.reshape(32, 48, 4225)

    x1 = jnp.pad(jnp.stack([s2d(context), s2d(body)]),
                 ((0, 0), (0, 0), (0, 0), (0, 7)))

    def w1pack(w):
        wp = jnp.pad(w, ((0, 0), (0, 0), (0, 1), (0, 1)))
        wr = wp.reshape(64, 3, 3, 4, 3, 4)
        wt = wr.transpose(2, 4, 0, 3, 5, 1)
        return wt.reshape(9, 64, 48).transpose(1, 0, 2).reshape(64, 432)

    w1 = jnp.stack([w1pack(cp['c1w']), w1pack(bp['c1w'])])
    b1 = jnp.stack([cp['c1b'][:, None], bp['c1b'][:, None]])
    offs1 = [a * 65 + b for a in range(3) for b in range(3)]
    y = _convT_pallas(x1, w1, b1, offs1, 4096)

    x2, _ = _lanes_to_grid(y, 65, 63, True, 2)
    taps2 = [(kh, kw) for kh in range(5) for kw in range(5)]
    w2 = _pack_w(cp['c2w'], bp['c2w'], taps2)
    b2 = jnp.stack([cp['c2b'][:, None], bp['c2b'][:, None]])
    offs2 = [kh * 35 + kw for kh, kw in taps2]
    y = _convT_pallas(x2, w2, b2, offs2, 1088)

    x3, _ = _lanes_to_grid(y, 35, 31, True, 1)
    taps3 = [(kh, kw) for kh in range(3) for kw in range(3)]
    offs3 = [kh * 17 + kw for kh, kw in taps3]
    w3 = _pack_w(cp['c3w'], bp['c3w'], taps3)
    b3 = jnp.stack([cp['c3b'][:, None], bp['c3b'][:, None]])
    y = _convT_pallas(x3, w3, b3, offs3, 256)

    x4, _ = _lanes_to_grid(y, 17, 15, False, 1)
    w4 = _pack_w(cp['c4w'], bp['c4w'], taps3)
    b4 = jnp.stack([cp['c4b'][:, None], bp['c4b'][:, None]])
    y = _convT_pallas(x4, w4, b4, offs3, 256)

    x5, _ = _lanes_to_grid(y, 17, 15, False, 1)
    w5 = _pack_w(cp['c5w'], bp['c5w'], taps3)
    b5 = jnp.stack([cp['c5b'][:, None], bp['c5b'][:, None]])
    y = _convT_pallas(x5, w5, b5, offs3, 256)

    y = jnp.pad(y, ((0, 0), (0, 0), (0, 0), (0, 33)))
    y = y.reshape(2, 32, 256, 17, 17)[:, :, :, :15, :15]
    y = lax.reduce_window(y, -jnp.inf, lax.max,
                          (1, 1, 1, 3, 3), (1, 1, 1, 2, 2), 'VALID')
    y = lax.reduce_window(y, 0.0, lax.add,
                          (1, 1, 1, 2, 2), (1, 1, 1, 1, 1), 'VALID') * 0.25
    feat = y.reshape(2, 32, 9216)
    return feat[0], feat[1]


def _conv(x, w, b, stride, pad):
    y = lax.conv_general_dilated(x, w, (stride, stride), [(pad, pad), (pad, pad)],
                                 dimension_numbers=('NCHW', 'OIHW', 'NCHW'))
    return y + b[None, :, None, None]


def _maxpool(x):
    return lax.reduce_window(x, -jnp.inf, lax.max, (1, 1, 3, 3), (1, 1, 2, 2), 'VALID')


def _alexnet_features(p, x):
    x = jax.nn.relu(_conv(x, p['c1w'], p['c1b'], 4, 2)); x = _maxpool(x)
    x = jax.nn.relu(_conv(x, p['c2w'], p['c2b'], 1, 2)); x = _maxpool(x)
    x = jax.nn.relu(_conv(x, p['c3w'], p['c3b'], 1, 1))
    x = jax.nn.relu(_conv(x, p['c4w'], p['c4b'], 1, 1))
    x = jax.nn.relu(_conv(x, p['c5w'], p['c5b'], 1, 1)); x = _maxpool(x)
    x = lax.reduce_window(x, 0.0, lax.add, (1, 1, 2, 2), (1, 1, 1, 1), 'VALID') * 0.25
    return x.reshape(x.shape[0], -1)


def kernel(x, colors, ctx_params, body_params, lstm_params, fc1_w, fc1_b, lift_w, lift_b):
    context = x[:, :, :256, :]
    body = x[:, :, 256:512, :]
    sem = x[:, :, 512:768, :]

    part = _hit_pallas(sem, colors)
    feat_sem = _lstm_pallas(part, lstm_params, lift_w, lift_b)

    featc = _alexnet_features(ctx_params, context)
    featb = _alexnet_features(body_params, body)
    fuse = _fc_head_pallas(featc, featb, ctx_params, body_params, fc1_w, fc1_b)
    return fuse, feat_sem


# speedup vs baseline: 4.3460x; 1.0005x over previous
import functools

import jax
import jax.numpy as jnp
from jax import lax
from jax.experimental import pallas as pl
from jax.experimental.pallas import tpu as pltpu

_NCPAD = 152
_NG = _NCPAD // 8


def _hit_kernel(cc_ref, sem_ref, out_ref, code_ref):
    j = pl.program_id(1)

    @pl.when(j == 0)
    def _():
        out_ref[...] = jnp.zeros_like(out_ref)

    s = sem_ref[0]
    code_ref[...] = s[0] * 65536.0 + s[1] * 256.0 + s[2]

    def chunk(r, _):
        tile = code_ref[pl.ds(r * 8, 8), :]
        px = [jnp.broadcast_to(tile[rr:rr + 1, :], (8, 256)) for rr in range(8)]
        for g in range(_NG):
            cc_g = cc_ref[g * 8:(g + 1) * 8, :]
            acc = out_ref[0, g * 8:(g + 1) * 8, :]
            for rr in range(8):
                acc = jnp.where(px[rr] == cc_g, 1.0, acc)
            out_ref[0, g * 8:(g + 1) * 8, :] = acc
        return 0

    lax.fori_loop(0, 32, chunk, 0)


def _hit_pallas(sem, colors):
    B = sem.shape[0]
    half = B // 2
    c = colors.astype(jnp.float32)
    ccode = c[:, 0] * 65536.0 + c[:, 1] * 256.0 + c[:, 2]
    ccode = jnp.concatenate([ccode, -jnp.ones((_NCPAD - 150,), jnp.float32)])
    cc_bc = jnp.broadcast_to(ccode[:, None], (_NCPAD, 256))

    out = pl.pallas_call(
        _hit_kernel,
        grid=(2, half),
        in_specs=[
            pl.BlockSpec((_NCPAD, 256), lambda c_, j: (0, 0)),
            pl.BlockSpec((1, 3, 256, 256), lambda c_, j, h=half: (c_ * h + j, 0, 0, 0)),
        ],
        out_specs=pl.BlockSpec((1, _NCPAD, 256), lambda c_, j: (c_, 0, 0)),
        out_shape=jax.ShapeDtypeStruct((2, _NCPAD, 256), jnp.float32),
        scratch_shapes=[pltpu.VMEM((256, 256), jnp.float32)],
        compiler_params=pltpu.CompilerParams(
            dimension_semantics=("parallel", "arbitrary")),
    )(cc_bc, sem)
    return out


def _lstm_kernel(part_ref, u0f_ref, u0b_ref, w0f_ref, w0b_ref,
                 bi1_ref, bh1_ref, wi1f_ref, wi1b_ref, w1f_ref, w1b_ref,
                 lwt_ref, lb_ref, out_ref, h1f_ref, h1b_ref, h2f_ref, h2b_ref):
    T = 32

    def cell(g, c):
        i = g[:, 0:16]
        f = g[:, 16:32]
        gg = g[:, 32:48]
        o = g[:, 48:64]
        c = jax.nn.sigmoid(f) * c + jax.nn.sigmoid(i) * jnp.tanh(gg)
        h = jax.nn.sigmoid(o) * jnp.tanh(c)
        return h, c

    def l1_scan(u, w_t, h_store, fwd):
        def body(k, carry):
            h, c = carry
            g = u + jnp.dot(h, w_t, preferred_element_type=jnp.float32)
            h, c = cell(g, c)
            idx = jnp.where(fwd, k, T - 1 - k)
            h_store[pl.ds(idx, 1), :, :] = h[None]
            return (h, c)
        lax.fori_loop(0, T, body, (jnp.zeros((8, 16), jnp.float32),
                                   jnp.zeros((8, 16), jnp.float32)))

    l1_scan(u0f_ref[...], w0f_ref[...], h1f_ref, True)
    l1_scan(u0b_ref[...], w0b_ref[...], h1b_ref, False)

    def l2_scan(wi_t, w_t, bi, bh, h_store, fwd):
        def body(k, carry):
            h, c = carry
            t = jnp.where(fwd, k, T - 1 - k)
            x2 = jnp.concatenate([h1f_ref[t], h1b_ref[t]], axis=1)
            u = (jnp.dot(x2, wi_t, preferred_element_type=jnp.float32) + bi) + bh
            g = u + jnp.dot(h, w_t, preferred_element_type=jnp.float32)
            h, c = cell(g, c)
            h_store[pl.ds(t, 1), :, :] = h[None]
            return (h, c)
        lax.fori_loop(0, T, body, (jnp.zeros((8, 16), jnp.float32),
                                   jnp.zeros((8, 16), jnp.float32)))

    l2_scan(wi1f_ref[0], w1f_ref[...], bi1_ref[0], bh1_ref[0], h2f_ref, True)
    l2_scan(wi1b_ref[0], w1b_ref[...], bi1_ref[1], bh1_ref[1], h2b_ref, False)

    hit2d = jnp.maximum(part_ref[0], part_ref[1])
    hcl = jnp.minimum(jnp.sum(hit2d, axis=1, keepdims=True), 1.0)
    z = jnp.sum(hcl * lwt_ref[...], axis=0, keepdims=True)
    s_row = jnp.sum(lwt_ref[...], axis=0, keepdims=True)

    msum = jnp.sum(h2f_ref[...], axis=2) + jnp.sum(h2b_ref[...], axis=2)
    m = msum / 32.0
    v0 = m[:, 0:1]
    v1 = m[:, 1:2]
    out_ref[...] = (lb_ref[...] + v0 * s_row) + (v1 - v0) * z


def _lstm_pallas(part, lstm_params, lift_w, lift_b):
    p = lstm_params
    vcol = (jnp.arange(8) == 1).astype(jnp.float32)[:, None]

    def u0(d):
        return (vcol * p['wih0'][d][:, 0][None, :] + p['bih0'][d][None, :]) \
            + p['bhh0'][d][None, :]

    lwt = jnp.pad(lift_w.T, ((0, 2), (0, 0)))
    out = pl.pallas_call(
        _lstm_kernel,
        out_shape=jax.ShapeDtypeStruct((32, 64), jnp.float32),
        scratch_shapes=[pltpu.VMEM((32, 8, 16), jnp.float32)] * 4,
    )(part, u0(0), u0(1), p['whh0'][0].T, p['whh0'][1].T,
      p['bih1'][:, None, :], p['bhh1'][:, None, :],
      p['wih1'][0].T[None], p['wih1'][1].T[None],
      p['whh1'][0].T, p['whh1'][1].T, lwt, lift_b[None, :])
    return out


def _mm_kernel(x_ref, w_ref, b_ref, o_ref, *, nsteps, relu):
    kb = pl.program_id(1)

    @pl.when(kb == 0)
    def _():
        o_ref[...] = jnp.broadcast_to(b_ref[...], o_ref.shape)

    o_ref[...] += lax.dot_general(
        x_ref[...], w_ref[...], (((1,), (1,)), ((), ())),
        preferred_element_type=jnp.float32)

    if relu:
        @pl.when(kb == nsteps - 1)
        def _():
            o_ref[...] = jnp.maximum(o_ref[...], 0.0)


def _mm_pallas(x, w, b, relu, kb_size):
    M, K = x.shape
    N = w.shape[0]
    nk = K // kb_size
    nh = N // 2
    return pl.pallas_call(
        functools.partial(_mm_kernel, nsteps=nk, relu=relu),
        grid=(2, nk),
        in_specs=[
            pl.BlockSpec((M, kb_size), lambda p, k: (0, k)),
            pl.BlockSpec((nh, kb_size), lambda p, k: (p, k)),
            pl.BlockSpec((1, nh), lambda p, k: (0, p)),
        ],
        out_specs=pl.BlockSpec((M, nh), lambda p, k: (0, p)),
        out_shape=jax.ShapeDtypeStruct((M, N), jnp.float32),
        compiler_params=pltpu.CompilerParams(
            dimension_semantics=("parallel", "arbitrary")),
    )(x, w, b)


def _fuse_kernel(x1_ref, x2_ref, w1_ref, w2_ref, b_ref, o_ref):
    y = lax.dot_general(x1_ref[...], w1_ref[...], (((1,), (1,)), ((), ())),
                        preferred_element_type=jnp.float32)
    y += lax.dot_general(x2_ref[...], w2_ref[...], (((1,), (1,)), ((), ())),
                         preferred_element_type=jnp.float32)
    o_ref[...] = y + b_ref[...]


def _fc_head_pallas(featc, featb, cp, bp, fc1_w, fc1_b):
    def fc_stack(p, x, n3_pad):
        x = _mm_pallas(x, p['f1w'], p['f1b'][None, :], True, 2304)
        x = _mm_pallas(x, p['f2w'], p['f2b'][None, :], True, 2048)
        w3 = jnp.pad(p['f3w'], ((0, n3_pad - p['f3w'].shape[0]), (0, 0)))
        b3 = jnp.pad(p['f3b'], (0, n3_pad - p['f3b'].shape[0]))
        return _mm_pallas(x, w3, b3[None, :], False, 2048)

    cf = fc_stack(cp, featc, 512)
    bf = fc_stack(bp, featb, 1024)
    wc = jnp.pad(fc1_w[:, :365], ((0, 0), (0, 512 - 365)))
    wb = jnp.pad(fc1_w[:, 365:], ((0, 0), (0, 1024 - 1000)))
    fuse = pl.pallas_call(
        _fuse_kernel,
        out_shape=jax.ShapeDtypeStruct((32, 512), jnp.float32),
    )(cf, bf, wc, wb, fc1_b[None, :])
    return fuse


def _convT_kernel(x_ref, w_ref, b_ref, o_ref, p_ref, *, cin, offs, nl):
    for i, off in enumerate(offs):
        p_ref[i * cin:(i + 1) * cin, :] = x_ref[0, 0, :, pl.ds(off, nl)]
    acc = jnp.dot(w_ref[0], p_ref[...], preferred_element_type=jnp.float32)
    o_ref[0, 0] = jnp.maximum(acc + b_ref[0], 0.0)


def _convT_pallas(x4, w, b, offs, nl):
    _, _, cin, spad = x4.shape
    cout, kg = w.shape[1], w.shape[2]
    return pl.pallas_call(
        functools.partial(_convT_kernel, cin=cin, offs=offs, nl=nl),
        grid=(2, 32),
        in_specs=[
            pl.BlockSpec((1, 1, cin, spad), lambda c, j: (c, j, 0, 0)),
            pl.BlockSpec((1, cout, kg), lambda c, j: (c, 0, 0)),
            pl.BlockSpec((1, cout, 1), lambda c, j: (c, 0, 0)),
        ],
        out_specs=pl.BlockSpec((1, 1, cout, nl), lambda c, j: (c, j, 0, 0)),
        out_shape=jax.ShapeDtypeStruct((2, 32, cout, nl), jnp.float32),
        scratch_shapes=[pltpu.VMEM((kg, nl), jnp.float32)],
        compiler_params=pltpu.CompilerParams(
            dimension_semantics=("parallel", "arbitrary")),
    )(x4, w, b)


def _pack_w(wc, wb, taps):
    def one(w):
        return jnp.concatenate([w[:, :, kh, kw] for kh, kw in taps], axis=1)
    return jnp.stack([one(wc), one(wb)])


def _lanes_to_grid(y, g, hv, pool, pad):
    c = y.shape[2]
    y = jnp.pad(y, ((0, 0), (0, 0), (0, 0), (0, g * g - y.shape[3])))
    y = y.reshape(2, 32, c, g, g)[:, :, :, :hv, :hv]
    if pool:
        y = lax.reduce_window(y, -jnp.inf, lax.max,
                              (1, 1, 1, 3, 3), (1, 1, 1, 2, 2), 'VALID')
    if pad:
        y = jnp.pad(y, ((0, 0), (0, 0), (0, 0), (pad, pad), (pad, pad)))
    s = y.shape[3]
    y = y.reshape(2, 32, c, s * s)
    lpad = (-(s * s) % 8) or 0
    return jnp.pad(y, ((0, 0), (0, 0), (0, 0), (0, lpad))), s


def _alexnet_convs_pallas(cp, bp, context, body):
    def s2d(x):
        xp = jnp.pad(x, ((0, 0), (0, 0), (2, 2), (2, 2)))
        xp = xp.reshape(32, 3, 65, 4, 65, 4)
        xp = xp.transpose(0, 3, 5, 1, 2, 4)
        return xp.reshape(32, 48, 4225)

    x1 = jnp.pad(jnp.stack([s2d(context), s2d(body)]),
                 ((0, 0), (0, 0), (0, 0), (0, 7)))

    def w1pack(w):
        wp = jnp.pad(w, ((0, 0), (0, 0), (0, 1), (0, 1)))
        wr = wp.reshape(64, 3, 3, 4, 3, 4)
        wt = wr.transpose(2, 4, 0, 3, 5, 1)
        return wt.reshape(9, 64, 48).transpose(1, 0, 2).reshape(64, 432)

    w1 = jnp.stack([w1pack(cp['c1w']), w1pack(bp['c1w'])])
    b1 = jnp.stack([cp['c1b'][:, None], bp['c1b'][:, None]])
    offs1 = [a * 65 + b for a in range(3) for b in range(3)]
    y = _convT_pallas(x1, w1, b1, offs1, 4096)

    x2, _ = _lanes_to_grid(y, 65, 63, True, 2)
    taps2 = [(kh, kw) for kh in range(5) for kw in range(5)]
    w2 = _pack_w(cp['c2w'], bp['c2w'], taps2)
    b2 = jnp.stack([cp['c2b'][:, None], bp['c2b'][:, None]])
    offs2 = [kh * 35 + kw for kh, kw in taps2]
    y = _convT_pallas(x2, w2, b2, offs2, 1088)

    x3, _ = _lanes_to_grid(y, 35, 31, True, 1)
    taps3 = [(kh, kw) for kh in range(3) for kw in range(3)]
    offs3 = [kh * 17 + kw for kh, kw in taps3]
    w3 = _pack_w(cp['c3w'], bp['c3w'], taps3)
    b3 = jnp.stack([cp['c3b'][:, None], bp['c3b'][:, None]])
    y = _convT_pallas(x3, w3, b3, offs3, 256)

    x4, _ = _lanes_to_grid(y, 17, 15, False, 1)
    w4 = _pack_w(cp['c4w'], bp['c4w'], taps3)
    b4 = jnp.stack([cp['c4b'][:, None], bp['c4b'][:, None]])
    y = _convT_pallas(x4, w4, b4, offs3, 256)

    x5, _ = _lanes_to_grid(y, 17, 15, False, 1)
    w5 = _pack_w(cp['c5w'], bp['c5w'], taps3)
    b5 = jnp.stack([cp['c5b'][:, None], bp['c5b'][:, None]])
    y = _convT_pallas(x5, w5, b5, offs3, 256)

    y = jnp.pad(y, ((0, 0), (0, 0), (0, 0), (0, 33)))
    y = y.reshape(2, 32, 256, 17, 17)[:, :, :, :15, :15]
    y = lax.reduce_window(y, -jnp.inf, lax.max,
                          (1, 1, 1, 3, 3), (1, 1, 1, 2, 2), 'VALID')
    y = lax.reduce_window(y, 0.0, lax.add,
                          (1, 1, 1, 2, 2), (1, 1, 1, 1, 1), 'VALID') * 0.25
    feat = y.reshape(2, 32, 9216)
    return feat[0], feat[1]


def _conv(x, w, b, stride, pad):
    y = lax.conv_general_dilated(x, w, (stride, stride), [(pad, pad), (pad, pad)],
                                 dimension_numbers=('NCHW', 'OIHW', 'NCHW'))
    return y + b[None, :, None, None]


def _maxpool(x):
    return lax.reduce_window(x, -jnp.inf, lax.max, (1, 1, 3, 3), (1, 1, 2, 2), 'VALID')


def _alexnet_features(p, x):
    x = jax.nn.relu(_conv(x, p['c1w'], p['c1b'], 4, 2)); x = _maxpool(x)
    x = jax.nn.relu(_conv(x, p['c2w'], p['c2b'], 1, 2)); x = _maxpool(x)
    x = jax.nn.relu(_conv(x, p['c3w'], p['c3b'], 1, 1))
    x = jax.nn.relu(_conv(x, p['c4w'], p['c4b'], 1, 1))
    x = jax.nn.relu(_conv(x, p['c5w'], p['c5b'], 1, 1)); x = _maxpool(x)
    x = lax.reduce_window(x, 0.0, lax.add, (1, 1, 2, 2), (1, 1, 1, 1), 'VALID') * 0.25
    return x.reshape(x.shape[0], -1)


def kernel(x, colors, ctx_params, body_params, lstm_params, fc1_w, fc1_b, lift_w, lift_b):
    context = x[:, :, :256, :]
    body = x[:, :, 256:512, :]
    sem = x[:, :, 512:768, :]

    part = _hit_pallas(sem, colors)
    feat_sem = _lstm_pallas(part, lstm_params, lift_w, lift_b)

    featc = _alexnet_features(ctx_params, context)
    featb = _alexnet_features(body_params, body)
    fuse = _fc_head_pallas(featc, featb, ctx_params, body_params, fc1_w, fc1_b)
    return fuse, feat_sem
